# Initial kernel scaffold; baseline (speedup 1.0000x reference)
import jax
import jax.numpy as jnp
from jax.experimental import pallas as pl


def kernel(extras, emb0, emb1, emb2, pe, W, b):
    raise NotImplementedError("write your pallas kernel here")


# speedup vs baseline: 3.7599x; 3.7599x over previous
import functools

import jax
import jax.numpy as jnp
from jax import lax
from jax.experimental import pallas as pl
from jax.experimental.pallas import tpu as pltpu
from jax.experimental.pallas import tpu_sc as plsc

D = 64
SEQ = 200
F32 = jnp.float32


def _proj_body(e0, e1, e2, pe, w, b, o0, o1, o2, opw):
    wt = w[:]
    dn = (((1,), (1,)), ((), ()))
    o0[:] = lax.dot_general(e0[:], wt[:, 0:64], dn, preferred_element_type=F32)
    o1[:] = lax.dot_general(e1[:], wt[:, 64:128], dn, preferred_element_type=F32)
    o2[:] = lax.dot_general(e2[:], wt[:, 128:192], dn, preferred_element_type=F32)

    @pl.when(pl.program_id(0) == 0)
    def _():
        opw[:] = lax.dot_general(pe[:], wt[:, 192:256], dn,
                                 preferred_element_type=F32) + b[:]


def _project(emb0, emb1, emb2, pe200, W, b2d):
    n = emb0.shape[0]
    blk = 1000
    grid = n // blk
    tbl_in = pl.BlockSpec((blk, D), lambda i: (i, 0))
    full = lambda shape: pl.BlockSpec(shape, lambda i: (0, 0))
    return pl.pallas_call(
        _proj_body,
        grid=(grid,),
        in_specs=[tbl_in, tbl_in, tbl_in,
                  full((SEQ, D)), full((D, 4 * D)), full((1, D))],
        out_specs=[tbl_in, tbl_in, tbl_in, full((SEQ, D))],
        out_shape=[jax.ShapeDtypeStruct((n, D), F32)] * 3
        + [jax.ShapeDtypeStruct((SEQ, D), F32)],
    )(emb0, emb1, emb2, pe200, W, b2d)


def _make_sc_gather(n_rows):
    info = plsc.get_sparse_core_info()
    nc, ns = info.num_cores, info.num_subcores
    nw = nc * ns
    seqs_per_w = n_rows // (nw * SEQ)

    mesh = plsc.VectorSubcoreMesh(core_axis_name="c", subcore_axis_name="s")

    @functools.partial(
        pl.kernel,
        mesh=mesh,
        compiler_params=pltpu.CompilerParams(use_tc_tiling_on_sc=False),
        out_type=[jax.ShapeDtypeStruct((n_rows, D), F32),
                  jax.ShapeDtypeStruct((n_rows, D), F32)],
        scratch_types=[
            pltpu.VMEM((SEQ,), jnp.int32),
            pltpu.VMEM((SEQ,), jnp.int32),
            pltpu.VMEM((SEQ,), jnp.int32),
            pltpu.VMEM((SEQ, D), F32),
            pltpu.VMEM((SEQ, D), F32),
            pltpu.VMEM((SEQ, D), F32),
            pltpu.VMEM((SEQ, D), F32),
            pltpu.VMEM((SEQ, D), F32),
            pltpu.SemaphoreType.DMA,
        ],
    )
    def sc_gather(ein0, ein1, ein2, etg0, etg1, etg2, p0, p1, p2, pew_hbm,
                  out_in, out_tg,
                  idx0, idx1, idx2, b0, b1, b2, out_v, pe_v, sem):
        wid = lax.axis_index("s") * nc + lax.axis_index("c")
        pltpu.sync_copy(pew_hbm, pe_v)

        for eh0, eh1, eh2, out_ref in ((ein0, ein1, ein2, out_in),
                                       (etg0, etg1, etg2, out_tg)):
            def chunk_body(c, _, eh0=eh0, eh1=eh1, eh2=eh2, out_ref=out_ref):
                base = (wid * seqs_per_w + c) * SEQ
                pltpu.sync_copy(eh0.at[pl.ds(base, SEQ)], idx0)
                pltpu.sync_copy(eh1.at[pl.ds(base, SEQ)], idx1)
                pltpu.sync_copy(eh2.at[pl.ds(base, SEQ)], idx2)
                descs = []
                for tbl, buf, idx in ((p0, b0, idx0), (p1, b1, idx1),
                                      (p2, b2, idx2)):
                    for lo, ln in ((0, 104), (104, 96)):
                        descs.append(pltpu.async_copy(
                            tbl.at[idx.at[pl.ds(lo, ln)]],
                            buf.at[pl.ds(lo, ln)], sem))
                for dsc in descs:
                    dsc.wait()

                def row_body(r, _):
                    for k in range(D // 16):
                        s = pl.ds(k * 16, 16)
                        out_v[r, s] = (b0[r, s] + b1[r, s] + b2[r, s]
                                       + pe_v[r, s])
                    return 0
                lax.fori_loop(0, SEQ, row_body, 0, unroll=2)

                pltpu.sync_copy(out_v, out_ref.at[pl.ds(base, SEQ)])
                return 0
            lax.fori_loop(0, seqs_per_w, chunk_body, 0)

    return sc_gather


def kernel(extras, emb0, emb1, emb2, pe, W, b):
    Bb, L = extras.shape[1], extras.shape[2]
    n_rows = Bb * L
    extras_flat = extras.reshape(6, n_rows).astype(jnp.int32)
    p0, p1, p2, pew = _project(emb0, emb1, emb2, pe[:SEQ], W,
                               b.reshape(1, D))
    sc = _make_sc_gather(n_rows)
    out_in, out_tg = sc(extras_flat[0], extras_flat[2], extras_flat[4],
                        extras_flat[1], extras_flat[3], extras_flat[5],
                        p0, p1, p2, pew)
    return out_in.reshape(Bb, L, D), out_tg.reshape(Bb, L, D)


# speedup vs baseline: 4.6448x; 1.2354x over previous
import functools

import jax
import jax.numpy as jnp
from jax import lax
from jax.experimental import pallas as pl
from jax.experimental.pallas import tpu as pltpu
from jax.experimental.pallas import tpu_sc as plsc

D = 64
SEQ = 200
F32 = jnp.float32


def _proj_body(e0, e1, e2, pe, w, b, o0, o1, o2, opw):
    wt = w[:]
    dn = (((1,), (1,)), ((), ()))
    o0[:] = lax.dot_general(e0[:], wt[:, 0:64], dn, preferred_element_type=F32)
    o1[:] = lax.dot_general(e1[:], wt[:, 64:128], dn, preferred_element_type=F32)
    o2[:] = lax.dot_general(e2[:], wt[:, 128:192], dn, preferred_element_type=F32)

    @pl.when(pl.program_id(0) == 0)
    def _():
        opw[:] = lax.dot_general(pe[:], wt[:, 192:256], dn,
                                 preferred_element_type=F32) + b[:]


def _project(emb0, emb1, emb2, pe200, W, b2d):
    n = emb0.shape[0]
    blk = 1000
    grid = n // blk
    tbl_in = pl.BlockSpec((blk, D), lambda i: (i, 0))
    full = lambda shape: pl.BlockSpec(shape, lambda i: (0, 0))
    return pl.pallas_call(
        _proj_body,
        grid=(grid,),
        in_specs=[tbl_in, tbl_in, tbl_in,
                  full((SEQ, D)), full((D, 4 * D)), full((1, D))],
        out_specs=[tbl_in, tbl_in, tbl_in, full((SEQ, D))],
        out_shape=[jax.ShapeDtypeStruct((n, D), F32)] * 3
        + [jax.ShapeDtypeStruct((SEQ, D), F32)],
    )(emb0, emb1, emb2, pe200, W, b2d)


def _make_sc_gather(n_rows):
    info = plsc.get_sparse_core_info()
    nc, ns = info.num_cores, info.num_subcores
    nw = nc * ns
    seqs_per_w = n_rows // (nw * SEQ)

    mesh = plsc.VectorSubcoreMesh(core_axis_name="c", subcore_axis_name="s")

    nseq = seqs_per_w

    @functools.partial(
        pl.kernel,
        mesh=mesh,
        compiler_params=pltpu.CompilerParams(use_tc_tiling_on_sc=False),
        out_type=[jax.ShapeDtypeStruct((n_rows, D), F32),
                  jax.ShapeDtypeStruct((n_rows, D), F32)],
        scratch_types=[
            pltpu.VMEM((SEQ * 32,), jnp.int32),
            pltpu.VMEM((SEQ * 32,), jnp.int32),
            pltpu.VMEM((SEQ * 32,), jnp.int32),
            pltpu.VMEM((SEQ, D), F32),
            pltpu.VMEM((SEQ, D), F32),
            pltpu.VMEM((SEQ, D), F32),
            pltpu.VMEM((SEQ, D), F32),
            pltpu.VMEM((SEQ, D), F32),
            pltpu.VMEM((SEQ, D), F32),
            pltpu.VMEM((SEQ, D), F32),
            pltpu.SemaphoreType.DMA,
            pltpu.SemaphoreType.DMA,
            pltpu.SemaphoreType.DMA,
            pltpu.SemaphoreType.DMA,
            pltpu.SemaphoreType.DMA,
        ],
    )
    def sc_gather(ein0, ein1, ein2, etg0, etg1, etg2, p0, p1, p2, pew_hbm,
                  out_in, out_tg,
                  idx0, idx1, idx2, a0, a1, a2, c0, c1, c2, pe_v,
                  idx_sem, gsemA, gsemB, osemA, osemB):
        wid = lax.axis_index("s") * nc + lax.axis_index("c")
        pltpu.sync_copy(pew_hbm, pe_v)
        slots = ((a0, a1, a2, gsemA, osemA), (c0, c1, c2, gsemB, osemB))
        tables = (p0, p1, p2)
        idxs = (idx0, idx1, idx2)
        sidebase = wid * nseq * SEQ

        def gather_descs(slot, c):
            bufs, gsem = slots[slot][:3], slots[slot][3]
            descs = []
            for t in range(3):
                for lo, ln in ((0, 104), (104, 96)):
                    descs.append(pltpu.make_async_copy(
                        tables[t].at[idxs[t].at[pl.ds(c * SEQ + lo, ln)]],
                        bufs[t].at[pl.ds(lo, ln)], gsem))
            return descs

        def out_desc(slot, c, out_ref):
            b = slots[slot][0]
            return pltpu.make_async_copy(
                b, out_ref.at[pl.ds(sidebase + c * SEQ, SEQ)],
                slots[slot][4])

        def compute(slot):
            b0, b1, b2 = slots[slot][:3]

            def row_body(r, _):
                for k in range(D // 16):
                    s = pl.ds(k * 16, 16)
                    b0[r, s] = b0[r, s] + b1[r, s] + b2[r, s] + pe_v[r, s]
                return 0
            lax.fori_loop(0, SEQ, row_body, 0, unroll=2)

        def steady_chunk(c, slot, out_ref):
            other = 1 - slot
            out_desc(other, 0, out_ref).wait()
            for dsc in gather_descs(other, c + 1):
                dsc.start()
            for dsc in gather_descs(slot, c):
                dsc.wait()
            compute(slot)
            out_desc(slot, c, out_ref).start()

        for eh0, eh1, eh2, out_ref in ((ein0, ein1, ein2, out_in),
                                       (etg0, etg1, etg2, out_tg)):
            dsc_i = [pltpu.make_async_copy(
                eh.at[pl.ds(sidebase, nseq * SEQ)], ix, idx_sem)
                for eh, ix in ((eh0, idx0), (eh1, idx1), (eh2, idx2))]
            for dsc in dsc_i:
                dsc.start()
            for dsc in dsc_i:
                dsc.wait()
            for dsc in gather_descs(0, 0):
                dsc.start()

            for dsc in gather_descs(1, 1):
                dsc.start()
            for dsc in gather_descs(0, 0):
                dsc.wait()
            compute(0)
            out_desc(0, 0, out_ref).start()

            def pair_body(i, _, out_ref=out_ref):
                steady_chunk(2 * i + 1, 1, out_ref)
                steady_chunk(2 * i + 2, 0, out_ref)
                return 0
            lax.fori_loop(0, (nseq - 2) // 2, pair_body, 0)

            out_desc(0, 0, out_ref).wait()
            for dsc in gather_descs(1, nseq - 1):
                dsc.wait()
            compute(1)
            out_desc(1, nseq - 1, out_ref).start()
            out_desc(1, 0, out_ref).wait()

    return sc_gather


def kernel(extras, emb0, emb1, emb2, pe, W, b):
    Bb, L = extras.shape[1], extras.shape[2]
    n_rows = Bb * L
    extras_flat = extras.reshape(6, n_rows).astype(jnp.int32)
    p0, p1, p2, pew = _project(emb0, emb1, emb2, pe[:SEQ], W,
                               b.reshape(1, D))
    sc = _make_sc_gather(n_rows)
    out_in, out_tg = sc(extras_flat[0], extras_flat[2], extras_flat[4],
                        extras_flat[1], extras_flat[3], extras_flat[5],
                        p0, p1, p2, pew)
    return out_in.reshape(Bb, L, D), out_tg.reshape(Bb, L, D)


# speedup vs baseline: 6.2961x; 1.3555x over previous
import functools

import jax
import jax.numpy as jnp
from jax import lax
from jax.experimental import pallas as pl
from jax.experimental.pallas import tpu as pltpu
from jax.experimental.pallas import tpu_sc as plsc

D = 64
SEQ = 200
F32 = jnp.float32


def _proj_body(e0, e1, e2, pe, w, b, o0, o1, o2, opw):
    wt = w[:]
    dn = (((1,), (1,)), ((), ()))
    o0[:] = lax.dot_general(e0[:], wt[:, 0:64], dn, preferred_element_type=F32)
    o1[:] = lax.dot_general(e1[:], wt[:, 64:128], dn, preferred_element_type=F32)
    o2[:] = lax.dot_general(e2[:], wt[:, 128:192], dn, preferred_element_type=F32)

    @pl.when(pl.program_id(0) == 0)
    def _():
        opw[:] = lax.dot_general(pe[:], wt[:, 192:256], dn,
                                 preferred_element_type=F32) + b[:]


def _project(emb0, emb1, emb2, pe200, W, b2d):
    n = emb0.shape[0]
    blk = 2000
    grid = n // blk
    tbl_in = pl.BlockSpec((blk, D), lambda i: (i, 0))
    full = lambda shape: pl.BlockSpec(shape, lambda i: (0, 0))
    return pl.pallas_call(
        _proj_body,
        grid=(grid,),
        in_specs=[tbl_in, tbl_in, tbl_in,
                  full((SEQ, D)), full((D, 4 * D)), full((1, D))],
        out_specs=[tbl_in, tbl_in, tbl_in, full((SEQ, D))],
        out_shape=[jax.ShapeDtypeStruct((n, D), F32)] * 3
        + [jax.ShapeDtypeStruct((SEQ, D), F32)],
    )(emb0, emb1, emb2, pe200, W, b2d)


def _make_sc_gather(n_rows):
    info = plsc.get_sparse_core_info()
    nc, ns = info.num_cores, info.num_subcores
    nw = nc * ns
    seqs_per_w = n_rows // (nw * SEQ)

    mesh = plsc.VectorSubcoreMesh(core_axis_name="c", subcore_axis_name="s")

    nseq = seqs_per_w

    @functools.partial(
        pl.kernel,
        mesh=mesh,
        compiler_params=pltpu.CompilerParams(use_tc_tiling_on_sc=False),
        out_type=[jax.ShapeDtypeStruct((n_rows // 2, 2 * D), F32),
                  jax.ShapeDtypeStruct((n_rows // 2, 2 * D), F32)],
        scratch_types=[
            pltpu.VMEM((SEQ * 32,), jnp.int32),
            pltpu.VMEM((SEQ * 32,), jnp.int32),
            pltpu.VMEM((SEQ * 32,), jnp.int32),
            pltpu.VMEM((SEQ, D), F32),
            pltpu.VMEM((SEQ, D), F32),
            pltpu.VMEM((SEQ, D), F32),
            pltpu.VMEM((SEQ, D), F32),
            pltpu.VMEM((SEQ, D), F32),
            pltpu.VMEM((SEQ, D), F32),
            pltpu.VMEM((SEQ, D), F32),
            pltpu.VMEM((SEQ // 2, 2 * D), F32),
            pltpu.SemaphoreType.DMA,
            pltpu.SemaphoreType.DMA,
            pltpu.SemaphoreType.DMA,
            pltpu.SemaphoreType.DMA,
        ],
    )
    def sc_gather(ein0, ein1, ein2, etg0, etg1, etg2, p0, p1, p2, pew_hbm,
                  out_in, out_tg,
                  idx0, idx1, idx2, a0, a1, a2, c0, c1, c2, pe_v, out_v,
                  idx_sem, gsemA, gsemB, out_sem):
        wid = lax.axis_index("s") * nc + lax.axis_index("c")
        pltpu.sync_copy(pew_hbm, pe_v)
        slots = ((a0, a1, a2, gsemA), (c0, c1, c2, gsemB))
        tables = (p0, p1, p2)
        idxs = (idx0, idx1, idx2)
        sidebase = wid * nseq * SEQ

        def gather_descs(slot, c):
            bufs, gsem = slots[slot][:3], slots[slot][3]
            descs = []
            for t in range(3):
                for lo, ln in ((0, 104), (104, 96)):
                    descs.append(pltpu.make_async_copy(
                        tables[t].at[idxs[t].at[pl.ds(c * SEQ + lo, ln)]],
                        bufs[t].at[pl.ds(lo, ln)], gsem))
            return descs

        def out_desc(c, out_ref):
            return pltpu.make_async_copy(
                out_v,
                out_ref.at[pl.ds((sidebase + c * SEQ) // 2, SEQ // 2)],
                out_sem)

        def compute(slot):
            b0, b1, b2 = slots[slot][:3]

            def row_body(j, _):
                for par in range(2):
                    r = 2 * j + par
                    for k in range(D // 16):
                        s = pl.ds(k * 16, 16)
                        so = pl.ds(par * D + k * 16, 16)
                        out_v[j, so] = (b0[r, s] + b1[r, s] + b2[r, s]
                                        + pe_v[r, s])
                return 0
            lax.fori_loop(0, SEQ // 2, row_body, 0)

        def steady_chunk(c, slot, out_ref):
            other = 1 - slot
            for dsc in gather_descs(other, c + 1):
                dsc.start()
            for dsc in gather_descs(slot, c):
                dsc.wait()
            out_desc(0, out_ref).wait()
            compute(slot)
            out_desc(c, out_ref).start()

        for eh0, eh1, eh2, out_ref in ((ein0, ein1, ein2, out_in),
                                       (etg0, etg1, etg2, out_tg)):
            dsc_i = [pltpu.make_async_copy(
                eh.at[pl.ds(sidebase, nseq * SEQ)], ix, idx_sem)
                for eh, ix in ((eh0, idx0), (eh1, idx1), (eh2, idx2))]
            for dsc in dsc_i:
                dsc.start()
            for dsc in dsc_i:
                dsc.wait()
            for dsc in gather_descs(0, 0):
                dsc.start()

            for dsc in gather_descs(1, 1):
                dsc.start()
            for dsc in gather_descs(0, 0):
                dsc.wait()
            compute(0)
            out_desc(0, out_ref).start()

            def pair_body(i, _, out_ref=out_ref):
                steady_chunk(2 * i + 1, 1, out_ref)
                steady_chunk(2 * i + 2, 0, out_ref)
                return 0
            lax.fori_loop(0, (nseq - 2) // 2, pair_body, 0)

            for dsc in gather_descs(1, nseq - 1):
                dsc.wait()
            out_desc(0, out_ref).wait()
            compute(1)
            out_desc(nseq - 1, out_ref).start()
            out_desc(0, out_ref).wait()

    return sc_gather


def kernel(extras, emb0, emb1, emb2, pe, W, b):
    Bb, L = extras.shape[1], extras.shape[2]
    n_rows = Bb * L
    extras_flat = extras.reshape(6, n_rows).astype(jnp.int32)
    p0, p1, p2, pew = _project(emb0, emb1, emb2, pe[:SEQ], W,
                               b.reshape(1, D))
    sc = _make_sc_gather(n_rows)
    out_in, out_tg = sc(extras_flat[0], extras_flat[2], extras_flat[4],
                        extras_flat[1], extras_flat[3], extras_flat[5],
                        p0, p1, p2, pew)
    return out_in.reshape(Bb, L, D), out_tg.reshape(Bb, L, D)
